# Initial kernel scaffold; baseline (speedup 1.0000x reference)
import jax
import jax.numpy as jnp
from jax.experimental import pallas as pl


def kernel(x, edge_index, edge_attr, batch, params):
    raise NotImplementedError("write your pallas kernel here")


# speedup vs baseline: 3.5491x; 3.5491x over previous
import functools

import jax
import jax.numpy as jnp
from jax import lax
from jax.experimental import pallas as pl
from jax.experimental.pallas import tpu as pltpu
from jax.experimental.pallas import tpu_sc as plsc

N = 10000
NP = 10240
D = 128
H = 256
HH = 128
G = 128
GP = 256
E = 320000
CH = 128
NS = 16
NC = 2
K_E = -(-E // (NS * CH))
EPT = K_E * CH
EP = EPT * NS
K_D = -(-E // (NC * NS * CH))
EP2 = K_D * CH * NC * NS
RPT = NP // NS
K_N = RPT // CH
GPT = GP // NS
CHN = 64
NPT2 = NP // (NC * NS)
K_G = NPT2 // CHN
NACC = NP + GP
APT = NACC // NS

_mesh = plsc.VectorSubcoreMesh(core_axis_name="c", subcore_axis_name="s")


def _f32(*shape):
    return jax.ShapeDtypeStruct(shape, jnp.float32)


@functools.partial(
    pl.kernel,
    out_type=_f32(NC * NP, HH),
    mesh=_mesh,
    scratch_types=[
        pltpu.VMEM_SHARED((NP, HH), jnp.float32),
        pltpu.VMEM((CH,), jnp.int32),
        pltpu.VMEM((1, CH), jnp.int32),
        pltpu.VMEM((CH, HH), jnp.float32),
        pltpu.SemaphoreType.DMA,
    ],
)
def _sc_spmm(h2, srcp2, dstp, zrows, acc2, acc_s, src_i, dst_i, rows, sem):
    c = lax.axis_index("c")
    s = lax.axis_index("s")
    pltpu.sync_copy(zrows, acc_s.at[pl.ds(s * RPT, RPT)])
    plsc.subcore_barrier()
    base = c * EP + s * EPT

    @pl.loop(0, K_E)
    def _chunk(j):
        off = pl.multiple_of(base + j * CH, 8)
        offd = pl.multiple_of(s * EPT + j * CH, 8)
        pltpu.sync_copy(srcp2.at[pl.ds(off, CH)], src_i)
        pltpu.sync_copy(dstp.at[pl.ds(offd, CH)], dst_i.at[0])
        pltpu.async_copy(h2.at[src_i], rows, sem).wait()
        pltpu.sync_copy(rows, acc_s.at[dst_i.at[0]], add=True)

    plsc.subcore_barrier()
    pltpu.sync_copy(acc_s.at[pl.ds(s * RPT, RPT)],
                    acc2.at[pl.ds(c * NP + s * RPT, RPT)])


@functools.partial(
    pl.kernel,
    out_type=[_f32(NC * NP, HH), _f32(NC * GP, HH)],
    mesh=_mesh,
    scratch_types=[
        pltpu.VMEM_SHARED((NACC, HH), jnp.float32),
        pltpu.VMEM((1, CH), jnp.int32),
        pltpu.VMEM((1, CHN), jnp.int32),
        pltpu.VMEM((CH, HH), jnp.float32),
    ],
)
def _sc_counts(dstp2, boffp, ones_h, zrows, degp, gcntp, acc_s, dst_i, bat_i,
               ones_v):
    c = lax.axis_index("c")
    s = lax.axis_index("s")
    pltpu.sync_copy(ones_h, ones_v)
    pltpu.sync_copy(zrows, acc_s.at[pl.ds(s * APT, RPT)])
    pltpu.sync_copy(zrows.at[pl.ds(0, APT - RPT)],
                    acc_s.at[pl.ds(s * APT + RPT, APT - RPT)])
    plsc.subcore_barrier()
    wid = c * NS + s
    base_e = wid * (K_D * CH)

    @pl.loop(0, K_D)
    def _echunk(j):
        off = pl.multiple_of(base_e + j * CH, 8)
        pltpu.sync_copy(dstp2.at[pl.ds(off, CH)], dst_i.at[0])
        pltpu.sync_copy(ones_v, acc_s.at[dst_i.at[0]], add=True)

    base_n = wid * NPT2

    @pl.loop(0, K_G)
    def _nchunk(j):
        off = pl.multiple_of(base_n + j * CHN, 8)
        pltpu.sync_copy(boffp.at[pl.ds(off, CHN)], bat_i.at[0])
        pltpu.sync_copy(ones_v.at[pl.ds(0, CHN)], acc_s.at[bat_i.at[0]],
                        add=True)

    plsc.subcore_barrier()
    pltpu.sync_copy(acc_s.at[pl.ds(s * RPT, RPT)],
                    degp.at[pl.ds(c * NP + s * RPT, RPT)])
    pltpu.sync_copy(acc_s.at[pl.ds(NP + s * GPT, GPT)],
                    gcntp.at[pl.ds(c * GP + s * GPT, GPT)])


@functools.partial(
    pl.kernel,
    out_type=_f32(NC * GP, HH),
    mesh=_mesh,
    scratch_types=[
        pltpu.VMEM_SHARED((GP, HH), jnp.float32),
        pltpu.VMEM((1, CH), jnp.int32),
        pltpu.VMEM((CH, HH), jnp.float32),
    ],
)
def _sc_pool(h2, batch_p, zrows, gsum2, acc_s, dst_i, rows):
    c = lax.axis_index("c")
    s = lax.axis_index("s")
    pltpu.sync_copy(zrows.at[pl.ds(0, GPT)], acc_s.at[pl.ds(s * GPT, GPT)])
    plsc.subcore_barrier()
    base = s * RPT

    @pl.loop(0, K_N)
    def _chunk(j):
        off = pl.multiple_of(base + j * CH, 8)
        offh = pl.multiple_of(c * NP + base + j * CH, 8)
        pltpu.sync_copy(h2.at[pl.ds(offh, CH)], rows)
        pltpu.sync_copy(batch_p.at[pl.ds(off, CH)], dst_i.at[0])
        pltpu.sync_copy(rows, acc_s.at[dst_i.at[0]], add=True)

    plsc.subcore_barrier()
    pltpu.sync_copy(acc_s.at[pl.ds(s * GPT, GPT)],
                    gsum2.at[pl.ds(c * GP + s * GPT, GPT)])


def _emb_body(x_ref, w_ref, b_ref, o_ref):
    o_ref[...] = (jnp.dot(x_ref[...], w_ref[...],
                          preferred_element_type=jnp.float32) + b_ref[...])


def _emb_tc(x_p, w, b):
    return pl.pallas_call(
        _emb_body,
        grid=(NC, NP // RPT),
        in_specs=[
            pl.BlockSpec((RPT, D), lambda h, i: (i, 0)),
            pl.BlockSpec((D, HH), lambda h, i: (0, h)),
            pl.BlockSpec((1, HH), lambda h, i: (0, h)),
        ],
        out_specs=pl.BlockSpec((RPT, HH), lambda h, i: (h * (NP // RPT) + i, 0)),
        out_shape=_f32(NC * NP, HH),
    )(x_p, w, b)


def _layer_body(do_relu, aa_ref, ab_ref, ha_ref, hb_ref, da_ref, db_ref,
                relw_ref, root_ref, b_ref, o_ref):
    r = 1.0 / jnp.maximum(da_ref[:, :1] + db_ref[:, :1], 1.0)
    out = (
        jnp.dot(aa_ref[...] * r, relw_ref[:HH, :],
                preferred_element_type=jnp.float32)
        + jnp.dot(ab_ref[...] * r, relw_ref[HH:, :],
                  preferred_element_type=jnp.float32)
        + jnp.dot(ha_ref[...], root_ref[:HH, :],
                  preferred_element_type=jnp.float32)
        + jnp.dot(hb_ref[...], root_ref[HH:, :],
                  preferred_element_type=jnp.float32)
        + b_ref[...]
    )
    if do_relu:
        out = jnp.maximum(out, 0.0)
    o_ref[...] = out


def _layer_tc(acc2, h2, degp, relw, root, b, do_relu):
    nb = NP // RPT
    return pl.pallas_call(
        functools.partial(_layer_body, do_relu),
        grid=(NC, nb),
        in_specs=[
            pl.BlockSpec((RPT, HH), lambda h, i: (i, 0)),
            pl.BlockSpec((RPT, HH), lambda h, i: (nb + i, 0)),
            pl.BlockSpec((RPT, HH), lambda h, i: (i, 0)),
            pl.BlockSpec((RPT, HH), lambda h, i: (nb + i, 0)),
            pl.BlockSpec((RPT, HH), lambda h, i: (i, 0)),
            pl.BlockSpec((RPT, HH), lambda h, i: (nb + i, 0)),
            pl.BlockSpec((H, HH), lambda h, i: (0, h)),
            pl.BlockSpec((H, HH), lambda h, i: (0, h)),
            pl.BlockSpec((1, HH), lambda h, i: (0, h)),
        ],
        out_specs=pl.BlockSpec((RPT, HH), lambda h, i: (h * nb + i, 0)),
        out_shape=_f32(NC * NP, HH),
    )(acc2, acc2, h2, h2, degp, degp, relw, root, b)


def _head_body(gs_ref, gc_ref, w0_ref, b0_ref, w1_ref, b1_ref, wmu_ref,
               wp_ref, bmu_ref, bp_ref, epsw_ref, epsb_ref, out_ref, kld_ref):
    rc = 1.0 / jnp.maximum(gc_ref[:G, :1] + gc_ref[GP:GP + G, :1], 1.0)
    ga = gs_ref[:G, :] * rc
    gb = gs_ref[GP:GP + G, :] * rc
    t = (jnp.dot(ga, w0_ref[:HH, :], preferred_element_type=jnp.float32)
         + jnp.dot(gb, w0_ref[HH:, :], preferred_element_type=jnp.float32)
         + b0_ref[...])
    t = jnp.maximum(t, 0.0)
    t = jnp.dot(t, w1_ref[...], preferred_element_type=jnp.float32) + b1_ref[...]
    t = jnp.maximum(t, 0.0)
    w_mu = wmu_ref[...]
    std_w = 1e-6 + jnp.log(1.0 + jnp.exp(wp_ref[...]))
    b_mu = bmu_ref[...]
    std_b = 1e-6 + jnp.log(1.0 + jnp.exp(bp_ref[...]))
    act_mu = jnp.dot(t, w_mu, preferred_element_type=jnp.float32)
    act_var = jnp.dot(t * t, std_w * std_w, preferred_element_type=jnp.float32)
    out_ref[...] = (act_mu + jnp.sqrt(act_var) * epsw_ref[...]
                    + (b_mu + std_b * epsb_ref[...]))
    kld_w = 0.5 * jnp.sum(2.0 * jnp.log(0.1 / std_w) - 1.0
                          + (std_w / 0.1) ** 2 + (w_mu / 0.1) ** 2)
    kld_b = 0.5 * jnp.sum(2.0 * jnp.log(0.1 / std_b) - 1.0
                          + (std_b / 0.1) ** 2 + (b_mu / 0.1) ** 2)
    kld_ref[...] = jnp.full((1, 1), kld_w + kld_b, jnp.float32)


def _head_tc(gsum2, gcnt2, w0, b0, w1, b1, w_mu, w_p, b_mu, b_p, eps_w,
             eps_b):
    return pl.pallas_call(
        _head_body,
        out_shape=[_f32(G, 1), _f32(1, 1)],
    )(gsum2, gcnt2, w0, b0, w1, b1, w_mu, w_p, b_mu, b_p, eps_w, eps_b)


def kernel(x, edge_index, edge_attr, batch, params):
    src = edge_index[0]
    dst = edge_index[1]
    srcp = jnp.concatenate([src, jnp.zeros((EP - E,), jnp.int32)])
    srcp2 = jnp.concatenate([srcp, srcp + NP])
    dstp = jnp.concatenate([dst, jnp.full((EP2 - E,), N, jnp.int32)])
    batch_p = jnp.concatenate([batch, jnp.full((NP - N,), G, jnp.int32)])
    boffp = batch_p + NP
    x_p = jnp.concatenate([x, jnp.zeros((NP - N, D), jnp.float32)])
    zrows = jnp.zeros((RPT, HH), jnp.float32)
    ones_h = jnp.ones((CH, HH), jnp.float32)
    eps_w = jax.random.normal(jax.random.key(1234), (G, 1), dtype=jnp.float32)
    eps_b = jax.random.normal(jax.random.key(5678), (1,),
                              dtype=jnp.float32).reshape(1, 1)

    h2 = _emb_tc(x_p, params['emb_W'], params['emb_b'].reshape(1, H))
    degp, gcntp = _sc_counts(dstp, boffp, ones_h, zrows)

    n_layers = len(params['gcn'])
    for i, p in enumerate(params['gcn']):
        acc2 = _sc_spmm(h2, srcp2, dstp, zrows)
        h2 = _layer_tc(acc2, h2, degp, p['relW'], p['root'],
                       p['bias'].reshape(1, H), i < n_layers - 1)

    gsum2 = _sc_pool(h2, batch_p, zrows)
    out, kld = _head_tc(
        gsum2, gcntp, params['lin0_W'], params['lin0_b'].reshape(1, H),
        params['lin1_W'], params['lin1_b'].reshape(1, H), params['w_mu'],
        params['w_p'], params['b_mu'].reshape(1, 1),
        params['b_p'].reshape(1, 1), eps_w, eps_b)
    return out, kld[0, 0], 0
